# Initial kernel scaffold; baseline (speedup 1.0000x reference)
import jax
import jax.numpy as jnp
from jax.experimental import pallas as pl


def kernel(x, s, hashed_indices):
    raise NotImplementedError("write your pallas kernel here")


# speedup vs baseline: 1.7133x; 1.7133x over previous
import jax
import jax.numpy as jnp
from jax import lax
from jax.experimental import pallas as pl
from jax.experimental.pallas import tpu as pltpu
from jax.experimental.pallas import tpu_sc as plsc

IN_DIM = 65536
OUT_DIM = 16384
COLS = 1024

NC = 2
NS = 16
L = 16

CW = 32
NPASS = COLS // (CW * NC)
RPT = IN_DIM // NS
NBLK = RPT // 128
CHUNK = 512
ZROWS = 256
OBLK = 256


def _body(x_hbm, s_hbm, idx_hbm, z_hbm, y_hbm,
          idx2_v, s_v, xchunk, obuf, nbuf, zbuf, acc):
    c = lax.axis_index("c")
    sid = lax.axis_index("s")
    rowbase = sid * RPT

    pltpu.sync_copy(z_hbm, zbuf)
    for b in range(NBLK):
        pltpu.sync_copy(idx_hbm.at[pl.ds(rowbase + b * 128, 128)],
                        idx2_v.at[b])
        pltpu.sync_copy(s_hbm.at[pl.ds(rowbase + b * 128, 128)],
                        s_v.at[b])

    for b in range(NBLK):
        for j in range(128 // L):
            iv = idx2_v[b, pl.ds(j * L, L)]
            sv = s_v[b, pl.ds(j * L, L)]
            bump = jnp.where(sv < 0.0, jnp.int32(OUT_DIM), jnp.int32(0))
            idx2_v[b, pl.ds(j * L, L)] = iv + bump

    def one_pass(p, _):
        coloff = (c * NPASS + p) * CW

        for z in range(2 * OUT_DIM // NS // ZROWS):
            pltpu.sync_copy(zbuf,
                            acc.at[pl.ds(sid * (2 * OUT_DIM // NS)
                                         + z * ZROWS, ZROWS)])
        plsc.subcore_barrier()

        for b2 in range(RPT // CHUNK):
            pltpu.sync_copy(
                x_hbm.at[pl.ds(rowbase + b2 * CHUNK, CHUNK),
                         pl.ds(coloff, CW)],
                xchunk)
            for b3 in range(CHUNK // 128):
                pltpu.sync_copy(xchunk.at[pl.ds(b3 * 128, 128)],
                                acc.at[idx2_v.at[b2 * (CHUNK // 128) + b3]],
                                add=True)
        plsc.subcore_barrier()

        orows = OUT_DIM // NS
        for ob in range(orows // OBLK):
            obase = sid * orows + ob * OBLK
            pltpu.sync_copy(acc.at[pl.ds(obase, OBLK)], obuf)
            pltpu.sync_copy(acc.at[pl.ds(OUT_DIM + obase, OBLK)], nbuf)

            def sub_row(r, _):
                for j in range(CW // L):
                    obuf[r, pl.ds(j * L, L)] = (obuf[r, pl.ds(j * L, L)]
                                                - nbuf[r, pl.ds(j * L, L)])
                return _
            lax.fori_loop(0, OBLK, sub_row, None)

            pltpu.sync_copy(obuf,
                            y_hbm.at[pl.ds(obase, OBLK),
                                     pl.ds(coloff, CW)])
        plsc.subcore_barrier()
        return _

    lax.fori_loop(0, NPASS, one_pass, None)


def kernel(x, s, hashed_indices):
    s1d = s.reshape(IN_DIM)
    zeros = jnp.zeros((ZROWS, CW), dtype=jnp.float32)
    mesh = plsc.VectorSubcoreMesh(core_axis_name="c", subcore_axis_name="s",
                                  num_cores=NC, num_subcores=NS)
    f = pl.kernel(
        _body,
        out_type=jax.ShapeDtypeStruct((OUT_DIM, COLS), jnp.float32),
        mesh=mesh,
        scratch_types=[
            pltpu.VMEM((NBLK, 128), jnp.int32),
            pltpu.VMEM((NBLK, 128), jnp.float32),
            pltpu.VMEM((CHUNK, CW), jnp.float32),
            pltpu.VMEM((OBLK, CW), jnp.float32),
            pltpu.VMEM((OBLK, CW), jnp.float32),
            pltpu.VMEM((ZROWS, CW), jnp.float32),
            pltpu.VMEM_SHARED((2 * OUT_DIM, CW), jnp.float32),
        ],
        compiler_params=pltpu.CompilerParams(use_tc_tiling_on_sc=False),
    )
    return f(x, s1d, hashed_indices, zeros)


# speedup vs baseline: 1.9251x; 1.1236x over previous
import jax
import jax.numpy as jnp
from jax import lax
from jax.experimental import pallas as pl
from jax.experimental.pallas import tpu as pltpu
from jax.experimental.pallas import tpu_sc as plsc

IN_DIM = 65536
OUT_DIM = 16384
COLS = 1024

NC = 2
NS = 16
L = 16

CW = 32
NPASS = COLS // (CW * NC)
RPT = IN_DIM // NS
NBLK = RPT // 128
CHUNK = 512
ZROWS = 128
OBLK = 256


def _body(x_hbm, s_hbm, idx_hbm, z_hbm, y_hbm,
          idx2_v, s_v, xbuf0, xbuf1, obuf, nbuf, zbuf, acc,
          sem_x0, sem_x1, sem_sc0, sem_sc1, sem_y):
    c = lax.axis_index("c")
    sid = lax.axis_index("s")
    rowbase = sid * RPT

    pltpu.sync_copy(z_hbm, zbuf)
    for b in range(NBLK):
        pltpu.sync_copy(idx_hbm.at[pl.ds(rowbase + b * 128, 128)],
                        idx2_v.at[b])
        pltpu.sync_copy(s_hbm.at[pl.ds(rowbase + b * 128, 128)],
                        s_v.at[b])

    for b in range(NBLK):
        for j in range(128 // L):
            iv = idx2_v[b, pl.ds(j * L, L)]
            sv = s_v[b, pl.ds(j * L, L)]
            bump = jnp.where(sv < 0.0, jnp.int32(OUT_DIM), jnp.int32(0))
            idx2_v[b, pl.ds(j * L, L)] = iv + bump

    def one_pass(p, _):
        coloff = (c * NPASS + p) * CW

        for z in range(2 * OUT_DIM // NS // ZROWS):
            pltpu.sync_copy(zbuf,
                            acc.at[pl.ds(sid * (2 * OUT_DIM // NS)
                                         + z * ZROWS, ZROWS)])
        plsc.subcore_barrier()

        xb = (xbuf0, xbuf1)
        sx = (sem_x0, sem_x1)
        ssc = (sem_sc0, sem_sc1)
        NCH = RPT // CHUNK
        SPC = CHUNK // 128

        def stage(b2):
            par = b2 % 2
            return pltpu.async_copy(
                x_hbm.at[pl.ds(rowbase + b2 * CHUNK, CHUNK),
                         pl.ds(coloff, CW)],
                xb[par], sx[par])

        xdescs = {0: stage(0)}
        scat_descs = {0: [], 1: []}
        for b2 in range(NCH):
            par = b2 % 2
            xdescs.pop(b2).wait()
            if b2 + 1 < NCH:
                npar = (b2 + 1) % 2
                for dsc in scat_descs[npar]:
                    dsc.wait()
                scat_descs[npar] = []
                xdescs[b2 + 1] = stage(b2 + 1)
            for b3 in range(SPC):
                scat_descs[par].append(pltpu.async_copy(
                    xb[par].at[pl.ds(b3 * 128, 128)],
                    acc.at[idx2_v.at[b2 * SPC + b3]],
                    ssc[par], add=True))
        for par in (0, 1):
            for dsc in scat_descs[par]:
                dsc.wait()
        plsc.subcore_barrier()

        orows = OUT_DIM // NS
        for ob in range(orows // OBLK):
            obase = sid * orows + ob * OBLK
            pltpu.sync_copy(acc.at[pl.ds(obase, OBLK)], obuf)
            pltpu.sync_copy(acc.at[pl.ds(OUT_DIM + obase, OBLK)], nbuf)

            def sub_row(r, _):
                for j in range(CW // L):
                    obuf[r, pl.ds(j * L, L)] = (obuf[r, pl.ds(j * L, L)]
                                                - nbuf[r, pl.ds(j * L, L)])
                return _
            lax.fori_loop(0, OBLK, sub_row, None)

            pltpu.sync_copy(obuf,
                            y_hbm.at[pl.ds(obase, OBLK),
                                     pl.ds(coloff, CW)])
        plsc.subcore_barrier()
        return _

    lax.fori_loop(0, NPASS, one_pass, None)


def kernel(x, s, hashed_indices):
    s1d = s.reshape(IN_DIM)
    zeros = jnp.zeros((ZROWS, CW), dtype=jnp.float32)
    mesh = plsc.VectorSubcoreMesh(core_axis_name="c", subcore_axis_name="s",
                                  num_cores=NC, num_subcores=NS)
    f = pl.kernel(
        _body,
        out_type=jax.ShapeDtypeStruct((OUT_DIM, COLS), jnp.float32),
        mesh=mesh,
        scratch_types=[
            pltpu.VMEM((NBLK, 128), jnp.int32),
            pltpu.VMEM((NBLK, 128), jnp.float32),
            pltpu.VMEM((CHUNK, CW), jnp.float32),
            pltpu.VMEM((CHUNK, CW), jnp.float32),
            pltpu.VMEM((OBLK, CW), jnp.float32),
            pltpu.VMEM((OBLK, CW), jnp.float32),
            pltpu.VMEM((ZROWS, CW), jnp.float32),
            pltpu.VMEM_SHARED((2 * OUT_DIM, CW), jnp.float32),
            pltpu.SemaphoreType.DMA,
            pltpu.SemaphoreType.DMA,
            pltpu.SemaphoreType.DMA,
            pltpu.SemaphoreType.DMA,
            pltpu.SemaphoreType.DMA,
        ],
        compiler_params=pltpu.CompilerParams(use_tc_tiling_on_sc=False),
    )
    return f(x, s1d, hashed_indices, zeros)
